# baseline (speedup 1.0000x reference)
import jax
import jax.numpy as jnp
from jax.experimental import pallas as pl

_Q = 16384
_K = 1000
_KPAD = 1024
_D = 128
_BQ = 2048
_CHUNKS = ((0, 384, 0, 336), (256, 768, 336, 672), (640, 1024, 672, _KPAD))


def _bf16_round(v):
    return v.astype(jnp.bfloat16).astype(jnp.float32)


def _kmeans_block(x_ref, c_ref, c2_ref, out_ref):
    x = x_ref[...]
    c = c_ref[...]
    dots = jax.lax.dot_general(
        x.astype(jnp.bfloat16), c.astype(jnp.bfloat16),
        (((1,), (1,)), ((), ())),
        preferred_element_type=jnp.float32,
    )
    x2 = jnp.sum(x * x, axis=1, keepdims=True)
    dist = (x2 + c2_ref[...]) + dots
    kr = jax.lax.broadcasted_iota(jnp.int32, (1, _KPAD), 1).astype(jnp.float32)

    acc_v = None
    acc_i = None
    for ls, le, lo, hi in _CHUNKS:
        d_s = dist[:, ls:le]
        kr_s = kr[:, ls:le]
        bias = jnp.where((kr_s >= lo) & (kr_s < hi), 0.0, jnp.inf)
        d_c = d_s + bias
        m = jnp.min(d_c, axis=1, keepdims=True)
        i = jnp.min(jnp.where(d_c <= m, kr_s, float(_KPAD)),
                    axis=1, keepdims=True)
        if acc_v is None:
            acc_v, acc_i = _bf16_round(m), i
        else:
            win = m < acc_v
            acc_i = jnp.where(win, i, acc_i)
            acc_v = jnp.where(win, _bf16_round(m), acc_v)
    out_ref[...] = acc_i.astype(jnp.int32).reshape(1, 1, _BQ)


def kernel(X, centers):
    c_pad = jnp.pad(centers * -2.0, ((0, _KPAD - _K), (0, 0)))
    c2 = jnp.pad(jnp.sum(centers * centers, axis=1), (0, _KPAD - _K),
                 constant_values=jnp.inf)[None, :]
    grid = _Q // _BQ
    out = pl.pallas_call(
        _kmeans_block,
        grid=(grid,),
        in_specs=[
            pl.BlockSpec((_BQ, _D), lambda i: (i, 0)),
            pl.BlockSpec((_KPAD, _D), lambda i: (0, 0)),
            pl.BlockSpec((1, _KPAD), lambda i: (0, 0)),
        ],
        out_specs=pl.BlockSpec((1, 1, _BQ), lambda i: (i, 0, 0)),
        out_shape=jax.ShapeDtypeStruct((grid, 1, _BQ), jnp.int32),
    )(X, c_pad, c2)
    return out.reshape(_Q)


# speedup vs baseline: 1.0255x; 1.0255x over previous
import jax
import jax.numpy as jnp
from jax.experimental import pallas as pl

_Q = 16384
_K = 1000
_KPAD = 1024
_D = 128
_BQ = 1024
_CHUNKS = (
    (0, 336, ((0, False), (1, False), (2, True))),
    (336, 672, ((2, True), (3, False), (4, False), (5, True))),
    (672, _KPAD, ((5, True), (6, False), (7, False))),
)


def _bf16_round(v):
    return v.astype(jnp.bfloat16).astype(jnp.float32)


def _kmeans_block(x_ref, c_ref, c2_ref, out_ref):
    x = x_ref[...]
    c = c_ref[...]
    dots = jax.lax.dot_general(
        x.astype(jnp.bfloat16), c.astype(jnp.bfloat16),
        (((1,), (1,)), ((), ())),
        preferred_element_type=jnp.float32,
    )
    x2 = jnp.sum(x * x, axis=1, keepdims=True)
    dist = (x2 + c2_ref[...]) + dots
    kr = jax.lax.broadcasted_iota(jnp.int32, (1, _KPAD), 1).astype(jnp.float32)

    acc_v = None
    acc_i = None
    for lo, hi, groups in _CHUNKS:
        vals = []
        krs = []
        for g, needs_mask in groups:
            dg = dist[:, g * 128:(g + 1) * 128]
            kg = kr[:, g * 128:(g + 1) * 128]
            if needs_mask:
                dg = dg + jnp.where((kg >= lo) & (kg < hi), 0.0, jnp.inf)
            vals.append(dg)
            krs.append(kg)
        comb = vals[0]
        for v in vals[1:]:
            comb = jnp.minimum(comb, v)
        m = jnp.min(comb, axis=1, keepdims=True)
        sel = jnp.where(vals[0] <= m, krs[0], float(_KPAD))
        for v, kg in zip(vals[1:], krs[1:]):
            sel = jnp.minimum(sel, jnp.where(v <= m, kg, float(_KPAD)))
        i = jnp.min(sel, axis=1, keepdims=True)
        if acc_v is None:
            acc_v, acc_i = _bf16_round(m), i
        else:
            win = m < acc_v
            acc_i = jnp.where(win, i, acc_i)
            acc_v = jnp.where(win, _bf16_round(m), acc_v)
    out_ref[...] = acc_i.astype(jnp.int32).reshape(1, 1, _BQ)


def kernel(X, centers):
    c_pad = jnp.pad(centers * -2.0, ((0, _KPAD - _K), (0, 0)))
    c2 = jnp.pad(jnp.sum(centers * centers, axis=1), (0, _KPAD - _K),
                 constant_values=jnp.inf)[None, :]
    grid = _Q // _BQ
    out = pl.pallas_call(
        _kmeans_block,
        grid=(grid,),
        in_specs=[
            pl.BlockSpec((_BQ, _D), lambda i: (i, 0)),
            pl.BlockSpec((_KPAD, _D), lambda i: (0, 0)),
            pl.BlockSpec((1, _KPAD), lambda i: (0, 0)),
        ],
        out_specs=pl.BlockSpec((1, 1, _BQ), lambda i: (i, 0, 0)),
        out_shape=jax.ShapeDtypeStruct((grid, 1, _BQ), jnp.int32),
    )(X, c_pad, c2)
    return out.reshape(_Q)
